# baseline (speedup 1.0000x reference)
import functools

import jax
import jax.numpy as jnp
from jax import lax
from jax.experimental import pallas as pl
from jax.experimental.pallas import tpu as pltpu
from jax.experimental.pallas import tpu_sc as plsc

C, D = 64, 768
CB = 8
LANES = 16
ROWS_PER_W = 2

_info = plsc.get_sparse_core_info()
_NC = _info.num_cores


def _bias_body(cht, rgt, bc, wm, bm, wcnt, bcnt, out, rows_v, small_v):
    wid = lax.axis_index("s") * _NC + lax.axis_index("c")
    base = wid * ROWS_PER_W
    pltpu.sync_copy(cht.at[pl.ds(base, ROWS_PER_W)], rows_v)
    pltpu.sync_copy(rgt.at[pl.ds(0, 1)], small_v.at[pl.ds(0, 1)])
    pltpu.sync_copy(bc, small_v.at[pl.ds(1, 1)])
    pltpu.sync_copy(wm, small_v.at[pl.ds(2, 1)])
    pltpu.sync_copy(bm, small_v.at[pl.ds(3, 1)])
    pltpu.sync_copy(wcnt, small_v.at[pl.ds(4, 1)])
    pltpu.sync_copy(bcnt, small_v.at[pl.ds(5, 1)])
    for j in range(D // LANES):
        ds = pl.ds(j * LANES, LANES)
        const = (small_v[0, ds] + small_v[1, ds] + small_v[2, ds]
                 + small_v[3, ds] + small_v[4, ds] + small_v[5, ds])
        for r in range(ROWS_PER_W):
            rows_v[r, ds] = rows_v[r, ds] + const
    pltpu.sync_copy(rows_v, out.at[pl.ds(base, ROWS_PER_W)])


_bias_sc = functools.partial(
    pl.kernel,
    mesh=plsc.VectorSubcoreMesh(core_axis_name="c", subcore_axis_name="s"),
    out_type=jax.ShapeDtypeStruct((C, D), jnp.float32),
    scratch_types=[
        pltpu.VMEM((ROWS_PER_W, D), jnp.float32),
        pltpu.VMEM((6, D), jnp.float32),
    ],
)(_bias_body)


def _add_body(x_ref, bias_ref, o_ref):
    o_ref[...] = x_ref[...] + bias_ref[...][None, :, None, :]


def kernel(x, channel_table, region_table, Wc, bc, Wm, bm, Wcnt, bcnt):
    B, Cx, T, Dx = x.shape
    del Wc

    bias = _bias_sc(
        channel_table,
        region_table[:1],
        bc.reshape(1, D),
        Wm.reshape(1, D),
        bm.reshape(1, D),
        Wcnt.reshape(1, D),
        bcnt.reshape(1, D),
    )

    out = pl.pallas_call(
        _add_body,
        grid=(B, Cx // CB),
        in_specs=[
            pl.BlockSpec((1, CB, T, Dx), lambda b, cb: (b, cb, 0, 0)),
            pl.BlockSpec((CB, Dx), lambda b, cb: (cb, 0)),
        ],
        out_specs=pl.BlockSpec((1, CB, T, Dx), lambda b, cb: (b, cb, 0, 0)),
        out_shape=jax.ShapeDtypeStruct((B, Cx, T, Dx), x.dtype),
    )(x, bias)
    return out


# speedup vs baseline: 1.0185x; 1.0185x over previous
import functools

import jax
import jax.numpy as jnp
from jax import lax
from jax.experimental import pallas as pl
from jax.experimental.pallas import tpu as pltpu
from jax.experimental.pallas import tpu_sc as plsc

C, D = 64, 768
CB = 8
LANES = 16
ROWS_PER_W = 2

_info = plsc.get_sparse_core_info()
_NC = _info.num_cores


def _bias_body(cht, rgt, bc, wm, bm, wcnt, bcnt, out, rows_v, small_v, sem):
    wid = lax.axis_index("s") * _NC + lax.axis_index("c")
    base = wid * ROWS_PER_W
    copies = [
        pltpu.async_copy(cht.at[pl.ds(base, ROWS_PER_W)], rows_v, sem),
        pltpu.async_copy(rgt.at[pl.ds(0, 1)], small_v.at[pl.ds(0, 1)], sem),
        pltpu.async_copy(bc, small_v.at[pl.ds(1, 1)], sem),
        pltpu.async_copy(wm, small_v.at[pl.ds(2, 1)], sem),
        pltpu.async_copy(bm, small_v.at[pl.ds(3, 1)], sem),
        pltpu.async_copy(wcnt, small_v.at[pl.ds(4, 1)], sem),
        pltpu.async_copy(bcnt, small_v.at[pl.ds(5, 1)], sem),
    ]
    for cp in copies:
        cp.wait()
    for j in range(D // LANES):
        ds = pl.ds(j * LANES, LANES)
        const = (small_v[0, ds] + small_v[1, ds] + small_v[2, ds]
                 + small_v[3, ds] + small_v[4, ds] + small_v[5, ds])
        for r in range(ROWS_PER_W):
            rows_v[r, ds] = rows_v[r, ds] + const
    pltpu.sync_copy(rows_v, out.at[pl.ds(base, ROWS_PER_W)])


_bias_sc = functools.partial(
    pl.kernel,
    mesh=plsc.VectorSubcoreMesh(core_axis_name="c", subcore_axis_name="s"),
    out_type=jax.ShapeDtypeStruct((C, D), jnp.float32),
    scratch_types=[
        pltpu.VMEM((ROWS_PER_W, D), jnp.float32),
        pltpu.VMEM((6, D), jnp.float32),
        pltpu.SemaphoreType.DMA,
    ],
)(_bias_body)


def _add_body(x_ref, bias_ref, o_ref):
    o_ref[...] = x_ref[...] + bias_ref[...][None, :, None, :]


def kernel(x, channel_table, region_table, Wc, bc, Wm, bm, Wcnt, bcnt):
    B, Cx, T, Dx = x.shape
    del Wc

    bias = _bias_sc(
        channel_table,
        region_table[:1],
        bc.reshape(1, D),
        Wm.reshape(1, D),
        bm.reshape(1, D),
        Wcnt.reshape(1, D),
        bcnt.reshape(1, D),
    )

    out = pl.pallas_call(
        _add_body,
        grid=(B, Cx // CB),
        in_specs=[
            pl.BlockSpec((1, CB, T, Dx), lambda b, cb: (b, cb, 0, 0)),
            pl.BlockSpec((CB, Dx), lambda b, cb: (cb, 0)),
        ],
        out_specs=pl.BlockSpec((1, CB, T, Dx), lambda b, cb: (b, cb, 0, 0)),
        out_shape=jax.ShapeDtypeStruct((B, Cx, T, Dx), x.dtype),
    )(x, bias)
    return out


# speedup vs baseline: 1.1255x; 1.1051x over previous
import jax
import jax.numpy as jnp
from jax.experimental import pallas as pl

CB = 2


def _body(x_ref, cht_ref, rgt_ref, bc_ref, wm_ref, bm_ref, wcnt_ref,
          bcnt_ref, o_ref):
    const = (rgt_ref[0, :] + bc_ref[0, :] + wm_ref[0, :] + bm_ref[0, :]
             + wcnt_ref[0, :] + bcnt_ref[0, :])
    bias = cht_ref[0] + const[None, :]
    o_ref[...] = x_ref[...] + bias[None, :, None, :]


def kernel(x, channel_table, region_table, Wc, bc, Wm, bm, Wcnt, bcnt):
    B, C, T, D = x.shape
    del Wc

    grid = (B, C // CB)
    small = lambda r, c: pl.BlockSpec((r, c), lambda b, cb: (0, 0))
    out = pl.pallas_call(
        _body,
        grid=grid,
        in_specs=[
            pl.BlockSpec((1, CB, T, D), lambda b, cb: (b, cb, 0, 0)),
            pl.BlockSpec((1, CB, D), lambda b, cb: (cb, 0, 0)),
            small(1, D),
            small(1, D),
            small(1, D),
            small(1, D),
            small(1, D),
            small(1, D),
        ],
        out_specs=pl.BlockSpec((1, CB, T, D), lambda b, cb: (b, cb, 0, 0)),
        out_shape=jax.ShapeDtypeStruct((B, C, T, D), x.dtype),
    )(
        x,
        channel_table.reshape(-1, CB, D),
        region_table[:1],
        bc.reshape(1, D),
        Wm.reshape(1, D),
        bm.reshape(1, D),
        Wcnt.reshape(1, D),
        bcnt.reshape(1, D),
    )
    return out
